# Initial kernel scaffold; baseline (speedup 1.0000x reference)
import jax
import jax.numpy as jnp
from jax.experimental import pallas as pl


def kernel(feat, edge_index, ln1_g, ln1_b, W_head, W_tail, W_ent, attn, ln2_g, ln2_b, ff_W1, ff_b1, ff_W2, ff_b2):
    raise NotImplementedError("write your pallas kernel here")


# speedup vs baseline: 58.7110x; 58.7110x over previous
import functools
import jax
import jax.numpy as jnp
from jax import lax
from jax.experimental import pallas as pl
from jax.experimental.pallas import tpu as pltpu
from jax.experimental.pallas import tpu_sc as plsc

N = 10000
E = 320000
D = 128
H = 8
DH = 16
HOPS = 5
ALPHA = 0.15
SLOPE = 0.2

NC = 2
NS = 16
NW = NC * NS
EPT = E // NW
CH = 80
NCHUNK = EPT // CH
RPT = 632
NP = NS * RPT

_mesh = plsc.VectorSubcoreMesh(core_axis_name="c", subcore_axis_name="s")


def _shuf(v, idx):
    dn = lax.GatherDimensionNumbers(offset_dims=(), collapsed_slice_dims=(0,),
                                    start_index_map=(0,))
    return lax.gather(v, idx[:, None], dn, slice_sizes=(1,),
                      mode=lax.GatherScatterMode.PROMISE_IN_BOUNDS)


@functools.partial(
    pl.kernel,
    mesh=_mesh,
    out_type=[
        jax.ShapeDtypeStruct((E, DH), jnp.float32),
        jax.ShapeDtypeStruct((NC, NP, DH), jnp.float32),
    ],
    scratch_types=[
        pltpu.VMEM((CH,), jnp.int32),
        pltpu.VMEM((CH,), jnp.int32),
        pltpu.VMEM((CH, D), jnp.float32),
        pltpu.VMEM((CH, D), jnp.float32),
        pltpu.VMEM((CH, DH), jnp.float32),
        pltpu.VMEM((D,), jnp.float32),
        pltpu.VMEM_SHARED((NP, DH), jnp.float32),
        pltpu.SemaphoreType.DMA,
        pltpu.SemaphoreType.DMA,
    ],
)
def _attn(head_hbm, tail_hbm, attn_hbm, src_hbm, dst_hbm, z16_hbm,
          w_hbm, denom_hbm,
          sidx, didx, hbuf, tbuf, wbuf, attn_v, dacc, sem1, sem2):
    c = lax.axis_index("c")
    s = lax.axis_index("s")
    wid = s * NC + c
    pltpu.sync_copy(attn_hbm, attn_v)
    pltpu.sync_copy(z16_hbm.at[pl.ds(s * RPT, RPT), :],
                    dacc.at[pl.ds(s * RPT, RPT), :])
    plsc.subcore_barrier()
    ebase = wid * EPT
    lane = lax.broadcasted_iota(jnp.int32, (DH,), 0)

    def chunk_body(i, carry):
        base = ebase + i * CH
        pltpu.sync_copy(src_hbm.at[pl.ds(base, CH)], sidx)
        pltpu.sync_copy(dst_hbm.at[pl.ds(base, CH)], didx)
        g1 = pltpu.async_copy(head_hbm.at[sidx], hbuf, sem1)
        g2 = pltpu.async_copy(tail_hbm.at[didx], tbuf, sem2)
        g1.wait()
        g2.wait()

        def edge_body(e, carry2):
            acc = jnp.zeros((DH,), jnp.float32)
            for h in range(H):
                sl = pl.ds(h * DH, DH)
                v = hbuf[e, sl] * tbuf[e, sl]
                v = jnp.maximum(v, SLOPE * v)
                v = v * attn_v[sl]
                for sh in (8, 4, 2, 1):
                    v = v + _shuf(v, lane ^ sh)
                acc = jnp.where(lane == h, jnp.maximum(v, 0.0), acc)
            wbuf[e, :] = acc
            return carry2

        lax.fori_loop(0, CH, edge_body, 0)
        pltpu.sync_copy(wbuf, w_hbm.at[pl.ds(base, CH), :])
        pltpu.sync_copy(wbuf, dacc.at[didx], add=True)
        return carry

    lax.fori_loop(0, NCHUNK, chunk_body, 0)
    plsc.subcore_barrier()
    pltpu.sync_copy(dacc.at[pl.ds(s * RPT, RPT), :],
                    denom_hbm.at[c, pl.ds(s * RPT, RPT), :])


@functools.partial(
    pl.kernel,
    mesh=_mesh,
    out_type=jax.ShapeDtypeStruct((NC, NP, D), jnp.float32),
    scratch_types=[
        pltpu.VMEM((CH,), jnp.int32),
        pltpu.VMEM((CH,), jnp.int32),
        pltpu.VMEM((CH, D), jnp.float32),
        pltpu.VMEM((CH, DH), jnp.float32),
        pltpu.VMEM_SHARED((NP, D), jnp.float32),
        pltpu.SemaphoreType.DMA,
    ],
)
def _hop(f_hbm, w_hbm, src_hbm, dst_hbm, z128_hbm,
         p_hbm,
         sidx, didx, fbuf, wbuf, facc, sem):
    c = lax.axis_index("c")
    s = lax.axis_index("s")
    wid = s * NC + c
    pltpu.sync_copy(z128_hbm.at[pl.ds(s * RPT, RPT), :],
                    facc.at[pl.ds(s * RPT, RPT), :])
    plsc.subcore_barrier()
    ebase = wid * EPT

    def chunk_body(i, carry):
        base = ebase + i * CH
        pltpu.sync_copy(src_hbm.at[pl.ds(base, CH)], sidx)
        pltpu.sync_copy(dst_hbm.at[pl.ds(base, CH)], didx)
        pltpu.sync_copy(w_hbm.at[pl.ds(base, CH), :], wbuf)
        pltpu.async_copy(f_hbm.at[sidx], fbuf, sem).wait()

        def edge_body(e, carry2):
            wrow = wbuf[e, :]
            for h in range(H):
                sl = pl.ds(h * DH, DH)
                fbuf[e, sl] = fbuf[e, sl] * wrow[h]
            return carry2

        lax.fori_loop(0, CH, edge_body, 0)
        pltpu.sync_copy(fbuf, facc.at[didx], add=True)
        return carry

    lax.fori_loop(0, NCHUNK, chunk_body, 0)
    plsc.subcore_barrier()
    pltpu.sync_copy(facc.at[pl.ds(s * RPT, RPT), :],
                    p_hbm.at[c, pl.ds(s * RPT, RPT), :])


_BN = 1000


def _pre_body(feat_ref, g_ref, b_ref, wcat_ref, head_ref, tail_ref, ent_ref):
    x = feat_ref[...]
    mu = jnp.mean(x, axis=1, keepdims=True)
    xc = x - mu
    var = jnp.mean(xc * xc, axis=1, keepdims=True)
    hn = xc * lax.rsqrt(var + 1e-5) * g_ref[...] + b_ref[...]
    hw = jnp.dot(hn, wcat_ref[...], preferred_element_type=jnp.float32)
    head_ref[...] = hw[:, :D]
    tail_ref[...] = hw[:, D:2 * D]
    ent_ref[...] = hw[:, 2 * D:]


def _pre(feat, g, b, wcat):
    return pl.pallas_call(
        _pre_body,
        grid=(N // _BN,),
        in_specs=[
            pl.BlockSpec((_BN, D), lambda i: (i, 0)),
            pl.BlockSpec((1, D), lambda i: (0, 0)),
            pl.BlockSpec((1, D), lambda i: (0, 0)),
            pl.BlockSpec((D, 3 * D), lambda i: (0, 0)),
        ],
        out_specs=[
            pl.BlockSpec((_BN, D), lambda i: (i, 0)),
            pl.BlockSpec((_BN, D), lambda i: (i, 0)),
            pl.BlockSpec((_BN, D), lambda i: (i, 0)),
        ],
        out_shape=[
            jax.ShapeDtypeStruct((N, D), jnp.float32),
            jax.ShapeDtypeStruct((N, D), jnp.float32),
            jax.ShapeDtypeStruct((N, D), jnp.float32),
        ],
    )(feat, g, b, wcat)


def _scaled_sum(p_ref, dnm_ref, ent_ref):
    sden = 1.0 / (dnm_ref[0][:, :H] + dnm_ref[1][:, :H] + 1e-16)
    sexp = lax.broadcast_in_dim(sden, (_BN, H, DH), (0, 1)).reshape(_BN, D)
    psum = p_ref[0] + p_ref[1]
    return (1.0 - ALPHA) * psum * sexp + ALPHA * ent_ref[...]


def _combine_body(p_ref, dnm_ref, ent_ref, f_ref):
    f_ref[...] = _scaled_sum(p_ref, dnm_ref, ent_ref)


def _combine(p, denom, ent):
    return pl.pallas_call(
        _combine_body,
        grid=(N // _BN,),
        in_specs=[
            pl.BlockSpec((NC, _BN, D), lambda i: (0, i, 0)),
            pl.BlockSpec((NC, _BN, DH), lambda i: (0, i, 0)),
            pl.BlockSpec((_BN, D), lambda i: (i, 0)),
        ],
        out_specs=pl.BlockSpec((_BN, D), lambda i: (i, 0)),
        out_shape=jax.ShapeDtypeStruct((N, D), jnp.float32),
    )(p, denom, ent)


def _post_body(p_ref, dnm_ref, ent_ref, feat_ref, g_ref, b_ref,
               w1_ref, b1_ref, w2_ref, b2_ref, out_ref):
    f = _scaled_sum(p_ref, dnm_ref, ent_ref)
    rst = f + feat_ref[...]
    mu = jnp.mean(rst, axis=1, keepdims=True)
    xc = rst - mu
    var = jnp.mean(xc * xc, axis=1, keepdims=True)
    ln = xc * lax.rsqrt(var + 1e-5) * g_ref[...] + b_ref[...]
    hid = jnp.dot(ln, w1_ref[...], preferred_element_type=jnp.float32)
    hid = jnp.maximum(hid + b1_ref[...], 0.0)
    ff = jnp.dot(hid, w2_ref[...], preferred_element_type=jnp.float32)
    out_ref[...] = ff + b2_ref[...] + rst


def _post(p, denom, ent, feat, g, b, w1, b1, w2, b2):
    return pl.pallas_call(
        _post_body,
        grid=(N // _BN,),
        in_specs=[
            pl.BlockSpec((NC, _BN, D), lambda i: (0, i, 0)),
            pl.BlockSpec((NC, _BN, DH), lambda i: (0, i, 0)),
            pl.BlockSpec((_BN, D), lambda i: (i, 0)),
            pl.BlockSpec((_BN, D), lambda i: (i, 0)),
            pl.BlockSpec((1, D), lambda i: (0, 0)),
            pl.BlockSpec((1, D), lambda i: (0, 0)),
            pl.BlockSpec((D, 4 * D), lambda i: (0, 0)),
            pl.BlockSpec((1, 4 * D), lambda i: (0, 0)),
            pl.BlockSpec((4 * D, D), lambda i: (0, 0)),
            pl.BlockSpec((1, D), lambda i: (0, 0)),
        ],
        out_specs=pl.BlockSpec((_BN, D), lambda i: (i, 0)),
        out_shape=jax.ShapeDtypeStruct((N, D), jnp.float32),
    )(p, denom, ent, feat, g, b, w1, b1, w2, b2)


def kernel(feat, edge_index, ln1_g, ln1_b, W_head, W_tail, W_ent, attn,
           ln2_g, ln2_b, ff_W1, ff_b1, ff_W2, ff_b2):
    src = edge_index[0].astype(jnp.int32)
    dst = edge_index[1].astype(jnp.int32)
    wcat = jnp.concatenate([W_head, W_tail, W_ent], axis=1)
    attn_flat = attn.reshape(D)
    z16 = jnp.zeros((NP, DH), jnp.float32)
    z128 = jnp.zeros((NP, D), jnp.float32)

    head, tail, ent = _pre(feat, ln1_g.reshape(1, D), ln1_b.reshape(1, D), wcat)
    w, denom = _attn(head, tail, attn_flat, src, dst, z16)

    f = ent
    for _ in range(HOPS - 1):
        p = _hop(f, w, src, dst, z128)
        f = _combine(p, denom, ent)
    p = _hop(f, w, src, dst, z128)
    return _post(p, denom, ent, feat, ln2_g.reshape(1, D), ln2_b.reshape(1, D),
                 ff_W1, ff_b1.reshape(1, 4 * D), ff_W2, ff_b2.reshape(1, D))


# speedup vs baseline: 59.1863x; 1.0081x over previous
import functools
import jax
import jax.numpy as jnp
from jax import lax
from jax.experimental import pallas as pl
from jax.experimental.pallas import tpu as pltpu
from jax.experimental.pallas import tpu_sc as plsc

N = 10000
E = 320000
D = 128
H = 8
DH = 16
HOPS = 5
ALPHA = 0.15
SLOPE = 0.2

NC = 2
NS = 16
NW = NC * NS
EPT = E // NW
CH = 80
NCHUNK = EPT // CH
NPAIR = (NCHUNK - 1) // 2
RPT = 632
NP = NS * RPT

_mesh = plsc.VectorSubcoreMesh(core_axis_name="c", subcore_axis_name="s")


def _shuf(v, idx):
    dn = lax.GatherDimensionNumbers(offset_dims=(), collapsed_slice_dims=(0,),
                                    start_index_map=(0,))
    return lax.gather(v, idx[:, None], dn, slice_sizes=(1,),
                      mode=lax.GatherScatterMode.PROMISE_IN_BOUNDS)


@functools.partial(
    pl.kernel,
    mesh=_mesh,
    out_type=[
        jax.ShapeDtypeStruct((E, DH), jnp.float32),
        jax.ShapeDtypeStruct((NC, NP, DH), jnp.float32),
    ],
    scratch_types=[
        pltpu.VMEM((CH,), jnp.int32),
        pltpu.VMEM((CH,), jnp.int32),
        pltpu.VMEM((CH,), jnp.int32),
        pltpu.VMEM((CH,), jnp.int32),
        pltpu.VMEM((CH, D), jnp.float32),
        pltpu.VMEM((CH, D), jnp.float32),
        pltpu.VMEM((CH, D), jnp.float32),
        pltpu.VMEM((CH, D), jnp.float32),
        pltpu.VMEM((CH, DH), jnp.float32),
        pltpu.VMEM((D,), jnp.float32),
        pltpu.VMEM_SHARED((NP, DH), jnp.float32),
        pltpu.SemaphoreType.DMA,
        pltpu.SemaphoreType.DMA,
        pltpu.SemaphoreType.DMA,
        pltpu.SemaphoreType.DMA,
    ],
)
def _attn(head_hbm, tail_hbm, attn_hbm, src_hbm, dst_hbm, z16_hbm,
          w_hbm, denom_hbm,
          sidx0, sidx1, didx0, didx1, hbuf0, hbuf1, tbuf0, tbuf1,
          wbuf, attn_v, dacc, semh0, semh1, semt0, semt1):
    c = lax.axis_index("c")
    s = lax.axis_index("s")
    wid = s * NC + c
    pltpu.sync_copy(attn_hbm, attn_v)
    pltpu.sync_copy(z16_hbm.at[pl.ds(s * RPT, RPT), :],
                    dacc.at[pl.ds(s * RPT, RPT), :])
    plsc.subcore_barrier()
    ebase = wid * EPT
    lane = lax.broadcasted_iota(jnp.int32, (DH,), 0)
    lo8 = lane < 8
    lo4 = (lane & 4) == 0
    lo2 = (lane & 2) == 0
    l7 = lane & 7
    perm = (((l7 & 1) << 2) | (l7 & 2) | ((l7 & 4) >> 2)) << 1

    def do_chunk(base, sidx, didx, hbuf, tbuf, wb, semh, semt):
        pltpu.sync_copy(src_hbm.at[pl.ds(base, CH)], sidx)
        pltpu.sync_copy(dst_hbm.at[pl.ds(base, CH)], didx)
        ch = pltpu.async_copy(head_hbm.at[sidx], hbuf, semh)
        ct = pltpu.async_copy(tail_hbm.at[didx], tbuf, semt)
        ch.wait()
        ct.wait()

        def edge_body(e, carry2):
            vs = []
            for h in range(H):
                sl = pl.ds(h * DH, DH)
                v = hbuf[e, sl] * tbuf[e, sl]
                v = jnp.maximum(v, SLOPE * v)
                vs.append(v * attn_v[sl])
            bs = []
            for j in range(4):
                x = vs[2 * j] + _shuf(vs[2 * j], lane ^ 8)
                y = vs[2 * j + 1] + _shuf(vs[2 * j + 1], lane ^ 8)
                bs.append(jnp.where(lo8, x, y))
            cs = []
            for j in range(2):
                x = bs[2 * j] + _shuf(bs[2 * j], lane ^ 4)
                y = bs[2 * j + 1] + _shuf(bs[2 * j + 1], lane ^ 4)
                cs.append(jnp.where(lo4, x, y))
            x = cs[0] + _shuf(cs[0], lane ^ 2)
            y = cs[1] + _shuf(cs[1], lane ^ 2)
            d = jnp.where(lo2, x, y)
            d = d + _shuf(d, lane ^ 1)
            acc = jnp.where(lo8, jnp.maximum(_shuf(d, perm), 0.0), 0.0)
            wb[e, :] = acc
            return carry2

        lax.fori_loop(0, CH, edge_body, 0)
        pltpu.sync_copy(wb, w_hbm.at[pl.ds(base, CH), :])
        pltpu.sync_copy(wb, dacc.at[didx], add=True)

    def chunk_body(i, carry):
        do_chunk(ebase + i * CH, sidx0, didx0, hbuf0, tbuf0, wbuf,
                 semh0, semt0)
        return carry

    lax.fori_loop(0, NCHUNK, chunk_body, 0)
    plsc.subcore_barrier()
    pltpu.sync_copy(dacc.at[pl.ds(s * RPT, RPT), :],
                    denom_hbm.at[c, pl.ds(s * RPT, RPT), :])


@functools.partial(
    pl.kernel,
    mesh=_mesh,
    out_type=jax.ShapeDtypeStruct((NC, NP, D), jnp.float32),
    scratch_types=[
        pltpu.VMEM((CH,), jnp.int32),
        pltpu.VMEM((CH,), jnp.int32),
        pltpu.VMEM((CH,), jnp.int32),
        pltpu.VMEM((CH,), jnp.int32),
        pltpu.VMEM((CH, D), jnp.float32),
        pltpu.VMEM((CH, D), jnp.float32),
        pltpu.VMEM((CH, DH), jnp.float32),
        pltpu.VMEM((CH, DH), jnp.float32),
        pltpu.VMEM_SHARED((NP, D), jnp.float32),
        pltpu.SemaphoreType.DMA,
        pltpu.SemaphoreType.DMA,
    ],
)
def _hop(f_hbm, w_hbm, src_hbm, dst_hbm, z128_hbm,
         p_hbm,
         sidx0, sidx1, didx0, didx1, fbuf0, fbuf1, wbuf0, wbuf1,
         facc, sem0, sem1):
    c = lax.axis_index("c")
    s = lax.axis_index("s")
    wid = s * NC + c
    pltpu.sync_copy(z128_hbm.at[pl.ds(s * RPT, RPT), :],
                    facc.at[pl.ds(s * RPT, RPT), :])
    plsc.subcore_barrier()
    ebase = wid * EPT
    lane = lax.broadcasted_iota(jnp.int32, (DH,), 0)
    hsel = [lane * 0 + h for h in range(H)]

    def do_chunk(base, sidx, didx, fbuf, wbuf, sem):
        pltpu.sync_copy(src_hbm.at[pl.ds(base, CH)], sidx)
        pltpu.sync_copy(dst_hbm.at[pl.ds(base, CH)], didx)
        pltpu.sync_copy(w_hbm.at[pl.ds(base, CH), :], wbuf)
        pltpu.async_copy(f_hbm.at[sidx], fbuf, sem).wait()

        def edge_body(e, carry2):
            wrow = wbuf[e, :]
            for h in range(H):
                sl = pl.ds(h * DH, DH)
                fbuf[e, sl] = fbuf[e, sl] * _shuf(wrow, hsel[h])
            return carry2

        lax.fori_loop(0, CH, edge_body, 0)
        pltpu.sync_copy(fbuf, facc.at[didx], add=True)

    def chunk_body(i, carry):
        do_chunk(ebase + i * CH, sidx0, didx0, fbuf0, wbuf0, sem0)
        return carry

    lax.fori_loop(0, NCHUNK, chunk_body, 0)
    plsc.subcore_barrier()
    pltpu.sync_copy(facc.at[pl.ds(s * RPT, RPT), :],
                    p_hbm.at[c, pl.ds(s * RPT, RPT), :])


_BN = 1000


def _pre_body(feat_ref, g_ref, b_ref, wcat_ref, head_ref, tail_ref, ent_ref):
    x = feat_ref[...]
    mu = jnp.mean(x, axis=1, keepdims=True)
    xc = x - mu
    var = jnp.mean(xc * xc, axis=1, keepdims=True)
    hn = xc * lax.rsqrt(var + 1e-5) * g_ref[...] + b_ref[...]
    hw = jnp.dot(hn, wcat_ref[...], preferred_element_type=jnp.float32)
    head_ref[...] = hw[:, :D]
    tail_ref[...] = hw[:, D:2 * D]
    ent_ref[...] = hw[:, 2 * D:]


def _pre(feat, g, b, wcat):
    return pl.pallas_call(
        _pre_body,
        grid=(N // _BN,),
        in_specs=[
            pl.BlockSpec((_BN, D), lambda i: (i, 0)),
            pl.BlockSpec((1, D), lambda i: (0, 0)),
            pl.BlockSpec((1, D), lambda i: (0, 0)),
            pl.BlockSpec((D, 3 * D), lambda i: (0, 0)),
        ],
        out_specs=[
            pl.BlockSpec((_BN, D), lambda i: (i, 0)),
            pl.BlockSpec((_BN, D), lambda i: (i, 0)),
            pl.BlockSpec((_BN, D), lambda i: (i, 0)),
        ],
        out_shape=[
            jax.ShapeDtypeStruct((N, D), jnp.float32),
            jax.ShapeDtypeStruct((N, D), jnp.float32),
            jax.ShapeDtypeStruct((N, D), jnp.float32),
        ],
    )(feat, g, b, wcat)


def _scaled_sum(p_ref, dnm_ref, ent_ref):
    sden = 1.0 / (dnm_ref[0][:, :H] + dnm_ref[1][:, :H] + 1e-16)
    sexp = lax.broadcast_in_dim(sden, (_BN, H, DH), (0, 1)).reshape(_BN, D)
    psum = p_ref[0] + p_ref[1]
    return (1.0 - ALPHA) * psum * sexp + ALPHA * ent_ref[...]


def _combine_body(p_ref, dnm_ref, ent_ref, f_ref):
    f_ref[...] = _scaled_sum(p_ref, dnm_ref, ent_ref)


def _combine(p, denom, ent):
    return pl.pallas_call(
        _combine_body,
        grid=(N // _BN,),
        in_specs=[
            pl.BlockSpec((NC, _BN, D), lambda i: (0, i, 0)),
            pl.BlockSpec((NC, _BN, DH), lambda i: (0, i, 0)),
            pl.BlockSpec((_BN, D), lambda i: (i, 0)),
        ],
        out_specs=pl.BlockSpec((_BN, D), lambda i: (i, 0)),
        out_shape=jax.ShapeDtypeStruct((N, D), jnp.float32),
    )(p, denom, ent)


def _post_body(p_ref, dnm_ref, ent_ref, feat_ref, g_ref, b_ref,
               w1_ref, b1_ref, w2_ref, b2_ref, out_ref):
    f = _scaled_sum(p_ref, dnm_ref, ent_ref)
    rst = f + feat_ref[...]
    mu = jnp.mean(rst, axis=1, keepdims=True)
    xc = rst - mu
    var = jnp.mean(xc * xc, axis=1, keepdims=True)
    ln = xc * lax.rsqrt(var + 1e-5) * g_ref[...] + b_ref[...]
    hid = jnp.dot(ln, w1_ref[...], preferred_element_type=jnp.float32)
    hid = jnp.maximum(hid + b1_ref[...], 0.0)
    ff = jnp.dot(hid, w2_ref[...], preferred_element_type=jnp.float32)
    out_ref[...] = ff + b2_ref[...] + rst


def _post(p, denom, ent, feat, g, b, w1, b1, w2, b2):
    return pl.pallas_call(
        _post_body,
        grid=(N // _BN,),
        in_specs=[
            pl.BlockSpec((NC, _BN, D), lambda i: (0, i, 0)),
            pl.BlockSpec((NC, _BN, DH), lambda i: (0, i, 0)),
            pl.BlockSpec((_BN, D), lambda i: (i, 0)),
            pl.BlockSpec((_BN, D), lambda i: (i, 0)),
            pl.BlockSpec((1, D), lambda i: (0, 0)),
            pl.BlockSpec((1, D), lambda i: (0, 0)),
            pl.BlockSpec((D, 4 * D), lambda i: (0, 0)),
            pl.BlockSpec((1, 4 * D), lambda i: (0, 0)),
            pl.BlockSpec((4 * D, D), lambda i: (0, 0)),
            pl.BlockSpec((1, D), lambda i: (0, 0)),
        ],
        out_specs=pl.BlockSpec((_BN, D), lambda i: (i, 0)),
        out_shape=jax.ShapeDtypeStruct((N, D), jnp.float32),
    )(p, denom, ent, feat, g, b, w1, b1, w2, b2)


def kernel(feat, edge_index, ln1_g, ln1_b, W_head, W_tail, W_ent, attn,
           ln2_g, ln2_b, ff_W1, ff_b1, ff_W2, ff_b2):
    src = edge_index[0].astype(jnp.int32)
    dst = edge_index[1].astype(jnp.int32)
    wcat = jnp.concatenate([W_head, W_tail, W_ent], axis=1)
    attn_flat = attn.reshape(D)
    z16 = jnp.zeros((NP, DH), jnp.float32)
    z128 = jnp.zeros((NP, D), jnp.float32)

    head, tail, ent = _pre(feat, ln1_g.reshape(1, D), ln1_b.reshape(1, D), wcat)
    w, denom = _attn(head, tail, attn_flat, src, dst, z16)

    f = ent
    for _ in range(HOPS - 1):
        p = _hop(f, w, src, dst, z128)
        f = _combine(p, denom, ent)
    p = _hop(f, w, src, dst, z128)
    return _post(p, denom, ent, feat, ln2_g.reshape(1, D), ln2_b.reshape(1, D),
                 ff_W1, ff_b1.reshape(1, 4 * D), ff_W2, ff_b2.reshape(1, D))


# speedup vs baseline: 59.2281x; 1.0007x over previous
import functools
import jax
import jax.numpy as jnp
from jax import lax
from jax.experimental import pallas as pl
from jax.experimental.pallas import tpu as pltpu
from jax.experimental.pallas import tpu_sc as plsc

N = 10000
E = 320000
D = 128
H = 8
DH = 16
HOPS = 5
ALPHA = 0.15
SLOPE = 0.2

NC = 2
NS = 16
NW = NC * NS
EPT = E // NW
CH = 80
NCHUNK = EPT // CH
RPT = 632
NP = NS * RPT

_mesh = plsc.VectorSubcoreMesh(core_axis_name="c", subcore_axis_name="s")


def _shuf(v, idx):
    dn = lax.GatherDimensionNumbers(offset_dims=(), collapsed_slice_dims=(0,),
                                    start_index_map=(0,))
    return lax.gather(v, idx[:, None], dn, slice_sizes=(1,),
                      mode=lax.GatherScatterMode.PROMISE_IN_BOUNDS)


@functools.partial(
    pl.kernel,
    mesh=_mesh,
    out_type=[
        jax.ShapeDtypeStruct((E, DH), jnp.float32),
        jax.ShapeDtypeStruct((NC, NP, DH), jnp.float32),
    ],
    scratch_types=[
        pltpu.VMEM((CH,), jnp.int32),
        pltpu.VMEM((CH,), jnp.int32),
        pltpu.VMEM((CH, D), jnp.float32),
        pltpu.VMEM((CH, D), jnp.float32),
        pltpu.VMEM((CH, DH), jnp.float32),
        pltpu.VMEM((D,), jnp.float32),
        pltpu.VMEM_SHARED((NP, DH), jnp.float32),
        pltpu.SemaphoreType.DMA,
        pltpu.SemaphoreType.DMA,
    ],
)
def _attn(head_hbm, tail_hbm, attn_hbm, src_hbm, dst_hbm, z16_hbm,
          w_hbm, denom_hbm,
          sidx, didx, hbuf, tbuf, wbuf, attn_v, dacc, semh, semt):
    c = lax.axis_index("c")
    s = lax.axis_index("s")
    wid = s * NC + c
    pltpu.sync_copy(attn_hbm, attn_v)
    pltpu.sync_copy(z16_hbm.at[pl.ds(s * RPT, RPT), :],
                    dacc.at[pl.ds(s * RPT, RPT), :])
    plsc.subcore_barrier()
    ebase = wid * EPT
    lane = lax.broadcasted_iota(jnp.int32, (DH,), 0)
    lo8 = lane < 8
    lo4 = (lane & 4) == 0
    lo2 = (lane & 2) == 0
    l7 = lane & 7
    perm = (((l7 & 1) << 2) | (l7 & 2) | ((l7 & 4) >> 2)) << 1

    def compute(wb):
        def edge_body(e, carry2):
            vs = []
            for h in range(H):
                sl = pl.ds(h * DH, DH)
                v = hbuf[e, sl] * tbuf[e, sl]
                v = jnp.maximum(v, SLOPE * v)
                vs.append(v * attn_v[sl])
            bs = []
            for j in range(4):
                x = vs[2 * j] + _shuf(vs[2 * j], lane ^ 8)
                y = vs[2 * j + 1] + _shuf(vs[2 * j + 1], lane ^ 8)
                bs.append(jnp.where(lo8, x, y))
            cs = []
            for j in range(2):
                x = bs[2 * j] + _shuf(bs[2 * j], lane ^ 4)
                y = bs[2 * j + 1] + _shuf(bs[2 * j + 1], lane ^ 4)
                cs.append(jnp.where(lo4, x, y))
            x = cs[0] + _shuf(cs[0], lane ^ 2)
            y = cs[1] + _shuf(cs[1], lane ^ 2)
            d = jnp.where(lo2, x, y)
            d = d + _shuf(d, lane ^ 1)
            acc = jnp.where(lo8, jnp.maximum(_shuf(d, perm), 0.0), 0.0)
            wb[e, :] = acc
            return carry2

        lax.fori_loop(0, CH, edge_body, 0)

    def chunk_body(i, carry):
        base = ebase + i * CH
        pltpu.sync_copy(src_hbm.at[pl.ds(base, CH)], sidx)
        pltpu.sync_copy(dst_hbm.at[pl.ds(base, CH)], didx)
        ch = pltpu.async_copy(head_hbm.at[sidx], hbuf, semh)
        ct = pltpu.async_copy(tail_hbm.at[didx], tbuf, semt)
        ch.wait()
        ct.wait()
        compute(wbuf)
        pltpu.sync_copy(wbuf, w_hbm.at[pl.ds(base, CH), :])
        pltpu.sync_copy(wbuf, dacc.at[didx], add=True)
        return carry

    lax.fori_loop(0, NCHUNK, chunk_body, 0)
    plsc.subcore_barrier()
    pltpu.sync_copy(dacc.at[pl.ds(s * RPT, RPT), :],
                    denom_hbm.at[c, pl.ds(s * RPT, RPT), :])


@functools.partial(
    pl.kernel,
    mesh=_mesh,
    out_type=jax.ShapeDtypeStruct((NC, NP, D), jnp.float32),
    scratch_types=[
        pltpu.VMEM((CH,), jnp.int32),
        pltpu.VMEM((CH,), jnp.int32),
        pltpu.VMEM((CH, D), jnp.float32),
        pltpu.VMEM((CH, DH), jnp.float32),
        pltpu.VMEM_SHARED((NP, D), jnp.float32),
        pltpu.SemaphoreType.DMA,
    ],
)
def _hop(f_hbm, w_hbm, src_hbm, dst_hbm, z128_hbm,
         p_hbm,
         sidx, didx, fbuf, wbuf, facc, sem):
    c = lax.axis_index("c")
    s = lax.axis_index("s")
    wid = s * NC + c
    pltpu.sync_copy(z128_hbm.at[pl.ds(s * RPT, RPT), :],
                    facc.at[pl.ds(s * RPT, RPT), :])
    plsc.subcore_barrier()
    ebase = wid * EPT
    lane = lax.broadcasted_iota(jnp.int32, (DH,), 0)
    hsel = [lane * 0 + h for h in range(H)]

    def compute():
        def edge_body(e, carry2):
            wrow = wbuf[e, :]
            for h in range(H):
                sl = pl.ds(h * DH, DH)
                fbuf[e, sl] = fbuf[e, sl] * _shuf(wrow, hsel[h])
            return carry2

        lax.fori_loop(0, CH, edge_body, 0)

    def chunk_body(i, carry):
        base = ebase + i * CH
        pltpu.sync_copy(src_hbm.at[pl.ds(base, CH)], sidx)
        pltpu.sync_copy(dst_hbm.at[pl.ds(base, CH)], didx)
        pltpu.sync_copy(w_hbm.at[pl.ds(base, CH), :], wbuf)
        pltpu.async_copy(f_hbm.at[sidx], fbuf, sem).wait()
        compute()
        pltpu.sync_copy(fbuf, facc.at[didx], add=True)
        return carry

    lax.fori_loop(0, NCHUNK, chunk_body, 0)
    plsc.subcore_barrier()
    pltpu.sync_copy(facc.at[pl.ds(s * RPT, RPT), :],
                    p_hbm.at[c, pl.ds(s * RPT, RPT), :])


_BN = 1000


def _pre_body(feat_ref, g_ref, b_ref, wcat_ref, head_ref, tail_ref, ent_ref):
    x = feat_ref[...]
    mu = jnp.mean(x, axis=1, keepdims=True)
    xc = x - mu
    var = jnp.mean(xc * xc, axis=1, keepdims=True)
    hn = xc * lax.rsqrt(var + 1e-5) * g_ref[...] + b_ref[...]
    hw = jnp.dot(hn, wcat_ref[...], preferred_element_type=jnp.float32)
    head_ref[...] = hw[:, :D]
    tail_ref[...] = hw[:, D:2 * D]
    ent_ref[...] = hw[:, 2 * D:]


def _pre(feat, g, b, wcat):
    return pl.pallas_call(
        _pre_body,
        grid=(N // _BN,),
        in_specs=[
            pl.BlockSpec((_BN, D), lambda i: (i, 0)),
            pl.BlockSpec((1, D), lambda i: (0, 0)),
            pl.BlockSpec((1, D), lambda i: (0, 0)),
            pl.BlockSpec((D, 3 * D), lambda i: (0, 0)),
        ],
        out_specs=[
            pl.BlockSpec((_BN, D), lambda i: (i, 0)),
            pl.BlockSpec((_BN, D), lambda i: (i, 0)),
            pl.BlockSpec((_BN, D), lambda i: (i, 0)),
        ],
        out_shape=[
            jax.ShapeDtypeStruct((N, D), jnp.float32),
            jax.ShapeDtypeStruct((N, D), jnp.float32),
            jax.ShapeDtypeStruct((N, D), jnp.float32),
        ],
    )(feat, g, b, wcat)


def _scaled_sum(p_ref, dnm_ref, ent_ref):
    sden = 1.0 / (dnm_ref[0][:, :H] + dnm_ref[1][:, :H] + 1e-16)
    sexp = lax.broadcast_in_dim(sden, (_BN, H, DH), (0, 1)).reshape(_BN, D)
    psum = p_ref[0] + p_ref[1]
    return (1.0 - ALPHA) * psum * sexp + ALPHA * ent_ref[...]


def _combine_body(p_ref, dnm_ref, ent_ref, f_ref):
    f_ref[...] = _scaled_sum(p_ref, dnm_ref, ent_ref)


def _combine(p, denom, ent):
    return pl.pallas_call(
        _combine_body,
        grid=(N // _BN,),
        in_specs=[
            pl.BlockSpec((NC, _BN, D), lambda i: (0, i, 0)),
            pl.BlockSpec((NC, _BN, DH), lambda i: (0, i, 0)),
            pl.BlockSpec((_BN, D), lambda i: (i, 0)),
        ],
        out_specs=pl.BlockSpec((_BN, D), lambda i: (i, 0)),
        out_shape=jax.ShapeDtypeStruct((N, D), jnp.float32),
    )(p, denom, ent)


def _post_body(p_ref, dnm_ref, ent_ref, feat_ref, g_ref, b_ref,
               w1_ref, b1_ref, w2_ref, b2_ref, out_ref):
    f = _scaled_sum(p_ref, dnm_ref, ent_ref)
    rst = f + feat_ref[...]
    mu = jnp.mean(rst, axis=1, keepdims=True)
    xc = rst - mu
    var = jnp.mean(xc * xc, axis=1, keepdims=True)
    ln = xc * lax.rsqrt(var + 1e-5) * g_ref[...] + b_ref[...]
    hid = jnp.dot(ln, w1_ref[...], preferred_element_type=jnp.float32)
    hid = jnp.maximum(hid + b1_ref[...], 0.0)
    ff = jnp.dot(hid, w2_ref[...], preferred_element_type=jnp.float32)
    out_ref[...] = ff + b2_ref[...] + rst


def _post(p, denom, ent, feat, g, b, w1, b1, w2, b2):
    return pl.pallas_call(
        _post_body,
        grid=(N // _BN,),
        in_specs=[
            pl.BlockSpec((NC, _BN, D), lambda i: (0, i, 0)),
            pl.BlockSpec((NC, _BN, DH), lambda i: (0, i, 0)),
            pl.BlockSpec((_BN, D), lambda i: (i, 0)),
            pl.BlockSpec((_BN, D), lambda i: (i, 0)),
            pl.BlockSpec((1, D), lambda i: (0, 0)),
            pl.BlockSpec((1, D), lambda i: (0, 0)),
            pl.BlockSpec((D, 4 * D), lambda i: (0, 0)),
            pl.BlockSpec((1, 4 * D), lambda i: (0, 0)),
            pl.BlockSpec((4 * D, D), lambda i: (0, 0)),
            pl.BlockSpec((1, D), lambda i: (0, 0)),
        ],
        out_specs=pl.BlockSpec((_BN, D), lambda i: (i, 0)),
        out_shape=jax.ShapeDtypeStruct((N, D), jnp.float32),
    )(p, denom, ent, feat, g, b, w1, b1, w2, b2)


def kernel(feat, edge_index, ln1_g, ln1_b, W_head, W_tail, W_ent, attn,
           ln2_g, ln2_b, ff_W1, ff_b1, ff_W2, ff_b2):
    src = edge_index[0].astype(jnp.int32)
    dst = edge_index[1].astype(jnp.int32)
    wcat = jnp.concatenate([W_head, W_tail, W_ent], axis=1)
    attn_flat = attn.reshape(D)
    z16 = jnp.zeros((NP, DH), jnp.float32)
    z128 = jnp.zeros((NP, D), jnp.float32)

    head, tail, ent = _pre(feat, ln1_g.reshape(1, D), ln1_b.reshape(1, D), wcat)
    w, denom = _attn(head, tail, attn_flat, src, dst, z16)

    f = ent
    for _ in range(HOPS - 1):
        p = _hop(f, w, src, dst, z128)
        f = _combine(p, denom, ent)
    p = _hop(f, w, src, dst, z128)
    return _post(p, denom, ent, feat, ln2_g.reshape(1, D), ln2_b.reshape(1, D),
                 ff_W1, ff_b1.reshape(1, 4 * D), ff_W2, ff_b2.reshape(1, D))


# speedup vs baseline: 63.3184x; 1.0691x over previous
import functools
import jax
import jax.numpy as jnp
from jax import lax
from jax.experimental import pallas as pl
from jax.experimental.pallas import tpu as pltpu
from jax.experimental.pallas import tpu_sc as plsc

N = 10000
E = 320000
D = 128
H = 8
DH = 16
HOPS = 5
ALPHA = 0.15
SLOPE = 0.2

NC = 2
NS = 16
NW = NC * NS
EPT = E // NW
CH = 80
NCHUNK = EPT // CH
WSS = 400
NSUPER = EPT // WSS
CPS = WSS // CH
NPR = CPS // 2
RPT = 632
NP = NS * RPT

_mesh = plsc.VectorSubcoreMesh(core_axis_name="c", subcore_axis_name="s")


def _shuf(v, idx):
    dn = lax.GatherDimensionNumbers(offset_dims=(), collapsed_slice_dims=(0,),
                                    start_index_map=(0,))
    return lax.gather(v, idx[:, None], dn, slice_sizes=(1,),
                      mode=lax.GatherScatterMode.PROMISE_IN_BOUNDS)


@functools.partial(
    pl.kernel,
    mesh=_mesh,
    out_type=[
        jax.ShapeDtypeStruct((E, DH), jnp.float32),
        jax.ShapeDtypeStruct((NC, NP, DH), jnp.float32),
    ],
    scratch_types=[
        pltpu.VMEM((CH,), jnp.int32),
        pltpu.VMEM((CH,), jnp.int32),
        pltpu.VMEM((CH, D), jnp.float32),
        pltpu.VMEM((CH, D), jnp.float32),
        pltpu.VMEM((CH, DH), jnp.float32),
        pltpu.VMEM((D,), jnp.float32),
        pltpu.VMEM_SHARED((NP, DH), jnp.float32),
        pltpu.SemaphoreType.DMA,
        pltpu.SemaphoreType.DMA,
    ],
)
def _attn(head_hbm, tail_hbm, attn_hbm, src_hbm, dst_hbm, z16_hbm,
          w_hbm, denom_hbm,
          sidx, didx, hbuf, tbuf, wbuf, attn_v, dacc, semh, semt):
    c = lax.axis_index("c")
    s = lax.axis_index("s")
    wid = s * NC + c
    pltpu.sync_copy(attn_hbm, attn_v)
    pltpu.sync_copy(z16_hbm.at[pl.ds(s * RPT, RPT), :],
                    dacc.at[pl.ds(s * RPT, RPT), :])
    plsc.subcore_barrier()
    ebase = wid * EPT
    lane = lax.broadcasted_iota(jnp.int32, (DH,), 0)
    lo8 = lane < 8
    lo4 = (lane & 4) == 0
    lo2 = (lane & 2) == 0
    l7 = lane & 7
    perm = (((l7 & 1) << 2) | (l7 & 2) | ((l7 & 4) >> 2)) << 1

    def compute():
        def edge_body(e, carry2):
            vs = []
            for h in range(H):
                sl = pl.ds(h * DH, DH)
                v = hbuf[e, sl] * tbuf[e, sl]
                v = jnp.maximum(v, SLOPE * v)
                vs.append(v * attn_v[sl])
            bs = []
            for j in range(4):
                x = vs[2 * j] + _shuf(vs[2 * j], lane ^ 8)
                y = vs[2 * j + 1] + _shuf(vs[2 * j + 1], lane ^ 8)
                bs.append(jnp.where(lo8, x, y))
            cs = []
            for j in range(2):
                x = bs[2 * j] + _shuf(bs[2 * j], lane ^ 4)
                y = bs[2 * j + 1] + _shuf(bs[2 * j + 1], lane ^ 4)
                cs.append(jnp.where(lo4, x, y))
            x = cs[0] + _shuf(cs[0], lane ^ 2)
            y = cs[1] + _shuf(cs[1], lane ^ 2)
            d = jnp.where(lo2, x, y)
            d = d + _shuf(d, lane ^ 1)
            acc = jnp.where(lo8, jnp.maximum(_shuf(d, perm), 0.0), 0.0)
            wbuf[e, :] = acc
            return carry2

        lax.fori_loop(0, CH, edge_body, 0)

    def chunk_body(i, carry):
        base = ebase + i * CH
        pltpu.sync_copy(src_hbm.at[pl.ds(base, CH)], sidx)
        pltpu.sync_copy(dst_hbm.at[pl.ds(base, CH)], didx)
        ch = pltpu.async_copy(head_hbm.at[sidx], hbuf, semh)
        ct = pltpu.async_copy(tail_hbm.at[didx], tbuf, semt)
        ch.wait()
        ct.wait()
        compute()
        pltpu.sync_copy(wbuf, w_hbm.at[pl.ds(base, CH), :])
        pltpu.sync_copy(wbuf, dacc.at[didx], add=True)
        return carry

    lax.fori_loop(0, NCHUNK, chunk_body, 0)
    plsc.subcore_barrier()
    pltpu.sync_copy(dacc.at[pl.ds(s * RPT, RPT), :],
                    denom_hbm.at[c, pl.ds(s * RPT, RPT), :])


@functools.partial(
    pl.kernel,
    mesh=_mesh,
    out_type=jax.ShapeDtypeStruct((NC, NP, D), jnp.float32),
    scratch_types=[
        pltpu.VMEM((CH,), jnp.int32),
        pltpu.VMEM((CH,), jnp.int32),
        pltpu.VMEM((CH,), jnp.int32),
        pltpu.VMEM((CH,), jnp.int32),
        pltpu.VMEM((CH, D), jnp.float32),
        pltpu.VMEM((CH, D), jnp.float32),
        pltpu.VMEM((CH, DH), jnp.float32),
        pltpu.VMEM((CH, DH), jnp.float32),
        pltpu.VMEM_SHARED((NP, D), jnp.float32),
        pltpu.SemaphoreType.DMA,
        pltpu.SemaphoreType.DMA,
    ],
)
def _hop(f_hbm, w_hbm, src_hbm, dst_hbm, z128_hbm,
         p_hbm,
         sidxa, didxa, sidxb, didxb, fbufa, fbufb, wbufa, wbufb,
         facc, sema, semb):
    c = lax.axis_index("c")
    s = lax.axis_index("s")
    wid = s * NC + c
    pltpu.sync_copy(z128_hbm.at[pl.ds(s * RPT, RPT), :],
                    facc.at[pl.ds(s * RPT, RPT), :])
    plsc.subcore_barrier()
    ebase = wid * EPT
    lane = lax.broadcasted_iota(jnp.int32, (DH,), 0)
    hsel = [lane * 0 + h for h in range(H)]

    def compute(fbuf, wbuf):
        def edge_body(e, carry2):
            wrow = wbuf[e, :]
            for h in range(H):
                sl = pl.ds(h * DH, DH)
                fbuf[e, sl] = fbuf[e, sl] * _shuf(wrow, hsel[h])
            return carry2

        lax.fori_loop(0, CH, edge_body, 0)

    def load_chunk(base, sidx, didx, wbuf):
        pltpu.sync_copy(src_hbm.at[pl.ds(base, CH)], sidx)
        pltpu.sync_copy(dst_hbm.at[pl.ds(base, CH)], didx)
        pltpu.sync_copy(w_hbm.at[pl.ds(base, CH), :], wbuf)

    def pair_body(p, carry):
        base_a = ebase + (2 * p) * CH
        base_b = base_a + CH
        load_chunk(base_a, sidxa, didxa, wbufa)
        load_chunk(base_b, sidxb, didxb, wbufb)
        ca = pltpu.async_copy(f_hbm.at[sidxa], fbufa, sema)
        cb = pltpu.async_copy(f_hbm.at[sidxb], fbufb, semb)
        ca.wait()
        cb.wait()
        compute(fbufa, wbufa)
        pltpu.sync_copy(fbufa, facc.at[didxa], add=True)
        compute(fbufb, wbufb)
        pltpu.sync_copy(fbufb, facc.at[didxb], add=True)
        return carry

    lax.fori_loop(0, NCHUNK // 2, pair_body, 0)
    base_z = ebase + (NCHUNK - 1) * CH
    load_chunk(base_z, sidxa, didxa, wbufa)
    pltpu.async_copy(f_hbm.at[sidxa], fbufa, sema).wait()
    compute(fbufa, wbufa)
    pltpu.sync_copy(fbufa, facc.at[didxa], add=True)
    plsc.subcore_barrier()
    pltpu.sync_copy(facc.at[pl.ds(s * RPT, RPT), :],
                    p_hbm.at[c, pl.ds(s * RPT, RPT), :])


_BN = 1000


def _pre_body(feat_ref, g_ref, b_ref, wcat_ref, head_ref, tail_ref, ent_ref):
    x = feat_ref[...]
    mu = jnp.mean(x, axis=1, keepdims=True)
    xc = x - mu
    var = jnp.mean(xc * xc, axis=1, keepdims=True)
    hn = xc * lax.rsqrt(var + 1e-5) * g_ref[...] + b_ref[...]
    hw = jnp.dot(hn, wcat_ref[...], preferred_element_type=jnp.float32)
    head_ref[...] = hw[:, :D]
    tail_ref[...] = hw[:, D:2 * D]
    ent_ref[...] = hw[:, 2 * D:]


def _pre(feat, g, b, wcat):
    return pl.pallas_call(
        _pre_body,
        grid=(N // _BN,),
        in_specs=[
            pl.BlockSpec((_BN, D), lambda i: (i, 0)),
            pl.BlockSpec((1, D), lambda i: (0, 0)),
            pl.BlockSpec((1, D), lambda i: (0, 0)),
            pl.BlockSpec((D, 3 * D), lambda i: (0, 0)),
        ],
        out_specs=[
            pl.BlockSpec((_BN, D), lambda i: (i, 0)),
            pl.BlockSpec((_BN, D), lambda i: (i, 0)),
            pl.BlockSpec((_BN, D), lambda i: (i, 0)),
        ],
        out_shape=[
            jax.ShapeDtypeStruct((N, D), jnp.float32),
            jax.ShapeDtypeStruct((N, D), jnp.float32),
            jax.ShapeDtypeStruct((N, D), jnp.float32),
        ],
    )(feat, g, b, wcat)


def _scaled_sum(p_ref, dnm_ref, ent_ref):
    sden = 1.0 / (dnm_ref[0][:, :H] + dnm_ref[1][:, :H] + 1e-16)
    sexp = lax.broadcast_in_dim(sden, (_BN, H, DH), (0, 1)).reshape(_BN, D)
    psum = p_ref[0] + p_ref[1]
    return (1.0 - ALPHA) * psum * sexp + ALPHA * ent_ref[...]


def _combine_body(p_ref, dnm_ref, ent_ref, f_ref):
    f_ref[...] = _scaled_sum(p_ref, dnm_ref, ent_ref)


def _combine(p, denom, ent):
    return pl.pallas_call(
        _combine_body,
        grid=(N // _BN,),
        in_specs=[
            pl.BlockSpec((NC, _BN, D), lambda i: (0, i, 0)),
            pl.BlockSpec((NC, _BN, DH), lambda i: (0, i, 0)),
            pl.BlockSpec((_BN, D), lambda i: (i, 0)),
        ],
        out_specs=pl.BlockSpec((_BN, D), lambda i: (i, 0)),
        out_shape=jax.ShapeDtypeStruct((N, D), jnp.float32),
    )(p, denom, ent)


def _post_body(p_ref, dnm_ref, ent_ref, feat_ref, g_ref, b_ref,
               w1_ref, b1_ref, w2_ref, b2_ref, out_ref):
    f = _scaled_sum(p_ref, dnm_ref, ent_ref)
    rst = f + feat_ref[...]
    mu = jnp.mean(rst, axis=1, keepdims=True)
    xc = rst - mu
    var = jnp.mean(xc * xc, axis=1, keepdims=True)
    ln = xc * lax.rsqrt(var + 1e-5) * g_ref[...] + b_ref[...]
    hid = jnp.dot(ln, w1_ref[...], preferred_element_type=jnp.float32)
    hid = jnp.maximum(hid + b1_ref[...], 0.0)
    ff = jnp.dot(hid, w2_ref[...], preferred_element_type=jnp.float32)
    out_ref[...] = ff + b2_ref[...] + rst


def _post(p, denom, ent, feat, g, b, w1, b1, w2, b2):
    return pl.pallas_call(
        _post_body,
        grid=(N // _BN,),
        in_specs=[
            pl.BlockSpec((NC, _BN, D), lambda i: (0, i, 0)),
            pl.BlockSpec((NC, _BN, DH), lambda i: (0, i, 0)),
            pl.BlockSpec((_BN, D), lambda i: (i, 0)),
            pl.BlockSpec((_BN, D), lambda i: (i, 0)),
            pl.BlockSpec((1, D), lambda i: (0, 0)),
            pl.BlockSpec((1, D), lambda i: (0, 0)),
            pl.BlockSpec((D, 4 * D), lambda i: (0, 0)),
            pl.BlockSpec((1, 4 * D), lambda i: (0, 0)),
            pl.BlockSpec((4 * D, D), lambda i: (0, 0)),
            pl.BlockSpec((1, D), lambda i: (0, 0)),
        ],
        out_specs=pl.BlockSpec((_BN, D), lambda i: (i, 0)),
        out_shape=jax.ShapeDtypeStruct((N, D), jnp.float32),
    )(p, denom, ent, feat, g, b, w1, b1, w2, b2)


def kernel(feat, edge_index, ln1_g, ln1_b, W_head, W_tail, W_ent, attn,
           ln2_g, ln2_b, ff_W1, ff_b1, ff_W2, ff_b2):
    src = edge_index[0].astype(jnp.int32)
    dst = edge_index[1].astype(jnp.int32)
    wcat = jnp.concatenate([W_head, W_tail, W_ent], axis=1)
    attn_flat = attn.reshape(D)
    z16 = jnp.zeros((NP, DH), jnp.float32)
    z128 = jnp.zeros((NP, D), jnp.float32)

    head, tail, ent = _pre(feat, ln1_g.reshape(1, D), ln1_b.reshape(1, D), wcat)
    w, denom = _attn(head, tail, attn_flat, src, dst, z16)

    f = ent
    for _ in range(HOPS - 1):
        p = _hop(f, w, src, dst, z128)
        f = _combine(p, denom, ent)
    p = _hop(f, w, src, dst, z128)
    return _post(p, denom, ent, feat, ln2_g.reshape(1, D), ln2_b.reshape(1, D),
                 ff_W1, ff_b1.reshape(1, 4 * D), ff_W2, ff_b2.reshape(1, D))


# speedup vs baseline: 67.6475x; 1.0684x over previous
import functools
import jax
import jax.numpy as jnp
from jax import lax
from jax.experimental import pallas as pl
from jax.experimental.pallas import tpu as pltpu
from jax.experimental.pallas import tpu_sc as plsc

N = 10000
E = 320000
D = 128
H = 8
DH = 16
HOPS = 5
ALPHA = 0.15
SLOPE = 0.2

NC = 2
NS = 16
NW = NC * NS
EPT = E // NW
CH = 80
NCHUNK = EPT // CH
WSS = 400
NSUPER = EPT // WSS
CPS = WSS // CH
NPR = CPS // 2
RPT = 632
NP = NS * RPT

_mesh = plsc.VectorSubcoreMesh(core_axis_name="c", subcore_axis_name="s")


def _shuf(v, idx):
    dn = lax.GatherDimensionNumbers(offset_dims=(), collapsed_slice_dims=(0,),
                                    start_index_map=(0,))
    return lax.gather(v, idx[:, None], dn, slice_sizes=(1,),
                      mode=lax.GatherScatterMode.PROMISE_IN_BOUNDS)


@functools.partial(
    pl.kernel,
    mesh=_mesh,
    out_type=[
        jax.ShapeDtypeStruct((E, DH), jnp.float32),
        jax.ShapeDtypeStruct((NC, NP, DH), jnp.float32),
    ],
    scratch_types=[
        pltpu.VMEM((CH,), jnp.int32),
        pltpu.VMEM((CH,), jnp.int32),
        pltpu.VMEM((CH, D), jnp.float32),
        pltpu.VMEM((CH, D), jnp.float32),
        pltpu.VMEM((CH, DH), jnp.float32),
        pltpu.VMEM((D,), jnp.float32),
        pltpu.VMEM_SHARED((NP, DH), jnp.float32),
        pltpu.SemaphoreType.DMA,
        pltpu.SemaphoreType.DMA,
    ],
)
def _attn(head_hbm, tail_hbm, attn_hbm, src_hbm, dst_hbm, z16_hbm,
          w_hbm, denom_hbm,
          sidx, didx, hbuf, tbuf, wbuf, attn_v, dacc, semh, semt):
    c = lax.axis_index("c")
    s = lax.axis_index("s")
    wid = s * NC + c
    pltpu.sync_copy(attn_hbm, attn_v)
    pltpu.sync_copy(z16_hbm.at[pl.ds(s * RPT, RPT), :],
                    dacc.at[pl.ds(s * RPT, RPT), :])
    plsc.subcore_barrier()
    ebase = wid * EPT
    lane = lax.broadcasted_iota(jnp.int32, (DH,), 0)
    lo8 = lane < 8
    lo4 = (lane & 4) == 0
    lo2 = (lane & 2) == 0
    l7 = lane & 7
    perm = (((l7 & 1) << 2) | (l7 & 2) | ((l7 & 4) >> 2)) << 1

    def compute():
        def edge_body(e, carry2):
            vs = []
            for h in range(H):
                sl = pl.ds(h * DH, DH)
                v = hbuf[e, sl] * tbuf[e, sl]
                v = jnp.maximum(v, SLOPE * v)
                vs.append(v * attn_v[sl])
            bs = []
            for j in range(4):
                x = vs[2 * j] + _shuf(vs[2 * j], lane ^ 8)
                y = vs[2 * j + 1] + _shuf(vs[2 * j + 1], lane ^ 8)
                bs.append(jnp.where(lo8, x, y))
            cs = []
            for j in range(2):
                x = bs[2 * j] + _shuf(bs[2 * j], lane ^ 4)
                y = bs[2 * j + 1] + _shuf(bs[2 * j + 1], lane ^ 4)
                cs.append(jnp.where(lo4, x, y))
            x = cs[0] + _shuf(cs[0], lane ^ 2)
            y = cs[1] + _shuf(cs[1], lane ^ 2)
            d = jnp.where(lo2, x, y)
            d = d + _shuf(d, lane ^ 1)
            acc = jnp.where(lo8, jnp.maximum(_shuf(d, perm), 0.0), 0.0)
            wbuf[e, :] = acc
            return carry2

        lax.fori_loop(0, CH, edge_body, 0)

    def chunk_body(i, carry):
        base = ebase + i * CH
        pltpu.sync_copy(src_hbm.at[pl.ds(base, CH)], sidx)
        pltpu.sync_copy(dst_hbm.at[pl.ds(base, CH)], didx)
        ch = pltpu.async_copy(head_hbm.at[sidx], hbuf, semh)
        ct = pltpu.async_copy(tail_hbm.at[didx], tbuf, semt)
        ch.wait()
        ct.wait()
        compute()
        pltpu.sync_copy(wbuf, w_hbm.at[pl.ds(base, CH), :])
        pltpu.sync_copy(wbuf, dacc.at[didx], add=True)
        return carry

    lax.fori_loop(0, NCHUNK, chunk_body, 0)
    plsc.subcore_barrier()
    pltpu.sync_copy(dacc.at[pl.ds(s * RPT, RPT), :],
                    denom_hbm.at[c, pl.ds(s * RPT, RPT), :])


@functools.partial(
    pl.kernel,
    mesh=_mesh,
    out_type=jax.ShapeDtypeStruct((NC, NP, D), jnp.float32),
    scratch_types=[
        pltpu.VMEM((CH,), jnp.int32),
        pltpu.VMEM((CH,), jnp.int32),
        pltpu.VMEM((CH,), jnp.int32),
        pltpu.VMEM((CH,), jnp.int32),
        pltpu.VMEM((CH, D), jnp.float32),
        pltpu.VMEM((CH, D), jnp.float32),
        pltpu.VMEM((2 * CH, DH), jnp.float32),
        pltpu.VMEM_SHARED((NP, D), jnp.float32),
        pltpu.SemaphoreType.DMA,
        pltpu.SemaphoreType.DMA,
    ],
)
def _hop(f_hbm, w_hbm, src_hbm, dst_hbm, z128_hbm,
         p_hbm,
         sidxa, didxa, sidxb, didxb, fbufa, fbufb, wpair,
         facc, sema, semb):
    c = lax.axis_index("c")
    s = lax.axis_index("s")
    wid = s * NC + c
    pltpu.sync_copy(z128_hbm.at[pl.ds(s * RPT, RPT), :],
                    facc.at[pl.ds(s * RPT, RPT), :])
    plsc.subcore_barrier()
    ebase = wid * EPT
    lane = lax.broadcasted_iota(jnp.int32, (DH,), 0)
    hsel = [lane * 0 + h for h in range(H)]

    def compute(fbuf, wo):
        def edge_body(e, carry2):
            wrow = wpair[wo + e, :]
            for h in range(H):
                sl = pl.ds(h * DH, DH)
                fbuf[e, sl] = fbuf[e, sl] * _shuf(wrow, hsel[h])
            return carry2

        lax.fori_loop(0, CH, edge_body, 0)

    def pair_body(p, carry):
        base_a = ebase + (2 * p) * CH
        base_b = base_a + CH
        pltpu.sync_copy(src_hbm.at[pl.ds(base_a, CH)], sidxa)
        pltpu.sync_copy(dst_hbm.at[pl.ds(base_a, CH)], didxa)
        pltpu.sync_copy(src_hbm.at[pl.ds(base_b, CH)], sidxb)
        pltpu.sync_copy(dst_hbm.at[pl.ds(base_b, CH)], didxb)
        pltpu.sync_copy(w_hbm.at[pl.ds(base_a, 2 * CH), :], wpair)
        ca = pltpu.async_copy(f_hbm.at[sidxa], fbufa, sema)
        cb = pltpu.async_copy(f_hbm.at[sidxb], fbufb, semb)
        ca.wait()
        cb.wait()
        compute(fbufa, 0)
        pltpu.sync_copy(fbufa, facc.at[didxa], add=True)
        compute(fbufb, CH)
        pltpu.sync_copy(fbufb, facc.at[didxb], add=True)
        return carry

    lax.fori_loop(0, NCHUNK // 2, pair_body, 0)
    base_z = ebase + (NCHUNK - 1) * CH
    pltpu.sync_copy(src_hbm.at[pl.ds(base_z, CH)], sidxa)
    pltpu.sync_copy(dst_hbm.at[pl.ds(base_z, CH)], didxa)
    pltpu.sync_copy(w_hbm.at[pl.ds(base_z, CH), :],
                    wpair.at[pl.ds(0, CH), :])
    pltpu.async_copy(f_hbm.at[sidxa], fbufa, sema).wait()
    compute(fbufa, 0)
    pltpu.sync_copy(fbufa, facc.at[didxa], add=True)
    plsc.subcore_barrier()
    pltpu.sync_copy(facc.at[pl.ds(s * RPT, RPT), :],
                    p_hbm.at[c, pl.ds(s * RPT, RPT), :])


_BN = 1000


def _pre_body(feat_ref, g_ref, b_ref, wcat_ref, head_ref, tail_ref, ent_ref):
    x = feat_ref[...]
    mu = jnp.mean(x, axis=1, keepdims=True)
    xc = x - mu
    var = jnp.mean(xc * xc, axis=1, keepdims=True)
    hn = xc * lax.rsqrt(var + 1e-5) * g_ref[...] + b_ref[...]
    hw = jnp.dot(hn, wcat_ref[...], preferred_element_type=jnp.float32)
    head_ref[...] = hw[:, :D]
    tail_ref[...] = hw[:, D:2 * D]
    ent_ref[...] = hw[:, 2 * D:]


def _pre(feat, g, b, wcat):
    return pl.pallas_call(
        _pre_body,
        grid=(N // _BN,),
        in_specs=[
            pl.BlockSpec((_BN, D), lambda i: (i, 0)),
            pl.BlockSpec((1, D), lambda i: (0, 0)),
            pl.BlockSpec((1, D), lambda i: (0, 0)),
            pl.BlockSpec((D, 3 * D), lambda i: (0, 0)),
        ],
        out_specs=[
            pl.BlockSpec((_BN, D), lambda i: (i, 0)),
            pl.BlockSpec((_BN, D), lambda i: (i, 0)),
            pl.BlockSpec((_BN, D), lambda i: (i, 0)),
        ],
        out_shape=[
            jax.ShapeDtypeStruct((N, D), jnp.float32),
            jax.ShapeDtypeStruct((N, D), jnp.float32),
            jax.ShapeDtypeStruct((N, D), jnp.float32),
        ],
    )(feat, g, b, wcat)


def _scaled_sum(p_ref, dnm_ref, ent_ref):
    sden = 1.0 / (dnm_ref[0][:, :H] + dnm_ref[1][:, :H] + 1e-16)
    sexp = lax.broadcast_in_dim(sden, (_BN, H, DH), (0, 1)).reshape(_BN, D)
    psum = p_ref[0] + p_ref[1]
    return (1.0 - ALPHA) * psum * sexp + ALPHA * ent_ref[...]


def _combine_body(p_ref, dnm_ref, ent_ref, f_ref):
    f_ref[...] = _scaled_sum(p_ref, dnm_ref, ent_ref)


def _combine(p, denom, ent):
    return pl.pallas_call(
        _combine_body,
        grid=(N // _BN,),
        in_specs=[
            pl.BlockSpec((NC, _BN, D), lambda i: (0, i, 0)),
            pl.BlockSpec((NC, _BN, DH), lambda i: (0, i, 0)),
            pl.BlockSpec((_BN, D), lambda i: (i, 0)),
        ],
        out_specs=pl.BlockSpec((_BN, D), lambda i: (i, 0)),
        out_shape=jax.ShapeDtypeStruct((N, D), jnp.float32),
    )(p, denom, ent)


def _post_body(p_ref, dnm_ref, ent_ref, feat_ref, g_ref, b_ref,
               w1_ref, b1_ref, w2_ref, b2_ref, out_ref):
    f = _scaled_sum(p_ref, dnm_ref, ent_ref)
    rst = f + feat_ref[...]
    mu = jnp.mean(rst, axis=1, keepdims=True)
    xc = rst - mu
    var = jnp.mean(xc * xc, axis=1, keepdims=True)
    ln = xc * lax.rsqrt(var + 1e-5) * g_ref[...] + b_ref[...]
    hid = jnp.dot(ln, w1_ref[...], preferred_element_type=jnp.float32)
    hid = jnp.maximum(hid + b1_ref[...], 0.0)
    ff = jnp.dot(hid, w2_ref[...], preferred_element_type=jnp.float32)
    out_ref[...] = ff + b2_ref[...] + rst


def _post(p, denom, ent, feat, g, b, w1, b1, w2, b2):
    return pl.pallas_call(
        _post_body,
        grid=(N // _BN,),
        in_specs=[
            pl.BlockSpec((NC, _BN, D), lambda i: (0, i, 0)),
            pl.BlockSpec((NC, _BN, DH), lambda i: (0, i, 0)),
            pl.BlockSpec((_BN, D), lambda i: (i, 0)),
            pl.BlockSpec((_BN, D), lambda i: (i, 0)),
            pl.BlockSpec((1, D), lambda i: (0, 0)),
            pl.BlockSpec((1, D), lambda i: (0, 0)),
            pl.BlockSpec((D, 4 * D), lambda i: (0, 0)),
            pl.BlockSpec((1, 4 * D), lambda i: (0, 0)),
            pl.BlockSpec((4 * D, D), lambda i: (0, 0)),
            pl.BlockSpec((1, D), lambda i: (0, 0)),
        ],
        out_specs=pl.BlockSpec((_BN, D), lambda i: (i, 0)),
        out_shape=jax.ShapeDtypeStruct((N, D), jnp.float32),
    )(p, denom, ent, feat, g, b, w1, b1, w2, b2)


def kernel(feat, edge_index, ln1_g, ln1_b, W_head, W_tail, W_ent, attn,
           ln2_g, ln2_b, ff_W1, ff_b1, ff_W2, ff_b2):
    src = edge_index[0].astype(jnp.int32)
    dst = edge_index[1].astype(jnp.int32)
    wcat = jnp.concatenate([W_head, W_tail, W_ent], axis=1)
    attn_flat = attn.reshape(D)
    z16 = jnp.zeros((NP, DH), jnp.float32)
    z128 = jnp.zeros((NP, D), jnp.float32)

    head, tail, ent = _pre(feat, ln1_g.reshape(1, D), ln1_b.reshape(1, D), wcat)
    w, denom = _attn(head, tail, attn_flat, src, dst, z16)

    f = ent
    for _ in range(HOPS - 1):
        p = _hop(f, w, src, dst, z128)
        f = _combine(p, denom, ent)
    p = _hop(f, w, src, dst, z128)
    return _post(p, denom, ent, feat, ln2_g.reshape(1, D), ln2_b.reshape(1, D),
                 ff_W1, ff_b1.reshape(1, 4 * D), ff_W2, ff_b2.reshape(1, D))
